# baseline (speedup 1.0000x reference)
import jax
import jax.numpy as jnp
from jax import lax
from jax.experimental import pallas as pl
from jax.experimental.pallas import tpu as pltpu

_BM = 4096
_NBUF = 4
_N = 32768
_NBLK = _N // _BM


def _copy_in(x_hbm, buf, sem, j, slot):
    pltpu.make_async_copy(
        x_hbm.at[pl.ds(j * _BM, _BM), :],
        buf.at[slot],
        sem.at[slot],
    ).start()


def _body(x_hbm, w_ref, out_ref, buf, sem):
    i = pl.program_id(0)

    @pl.when(i == 0)
    def _():
        for j in range(min(_NBUF, _NBLK)):
            _copy_in(x_hbm, buf, sem, j, j)

    @pl.when(jnp.logical_and(i > 0, i + _NBUF - 1 < _NBLK))
    def _():
        j = i + _NBUF - 1
        _copy_in(x_hbm, buf, sem, j, lax.rem(j, _NBUF))

    slot = lax.rem(i, _NBUF)
    pltpu.make_async_copy(
        x_hbm.at[pl.ds(i * _BM, _BM), :],
        buf.at[slot],
        sem.at[slot],
    ).wait()

    logits = jnp.dot(buf[slot], w_ref[...], preferred_element_type=jnp.float32)
    lt = logits.T
    m = jnp.max(lt, axis=0, keepdims=True)
    e = jnp.exp(lt - m)
    out_ref[...] = e / jnp.sum(e, axis=0, keepdims=True)


@jax.jit
def kernel(x, w_gate, w_noise):
    n, d = x.shape
    _, k = w_gate.shape
    out_t = pl.pallas_call(
        _body,
        grid=(_NBLK,),
        in_specs=[
            pl.BlockSpec(memory_space=pltpu.HBM),
            pl.BlockSpec((d, k), lambda i: (0, 0)),
        ],
        out_specs=pl.BlockSpec((k, _BM), lambda i: (0, i)),
        out_shape=jax.ShapeDtypeStruct((k, n), jnp.float32),
        scratch_shapes=[
            pltpu.VMEM((_NBUF, _BM, d), jnp.float32),
            pltpu.SemaphoreType.DMA((_NBUF,)),
        ],
    )(x, w_gate)
    return out_t.T


# speedup vs baseline: 1.0481x; 1.0481x over previous
import jax
import jax.numpy as jnp
from jax.experimental import pallas as pl
from jax.experimental.pallas import tpu as pltpu

_BM = 4096


def _body(x_ref, w_ref, out_ref):
    logits = jnp.dot(x_ref[...], w_ref[...], preferred_element_type=jnp.float32)
    lt = logits.T
    m = jnp.max(lt, axis=0, keepdims=True)
    e = jnp.exp(lt - m)
    out_ref[...] = e / jnp.sum(e, axis=0, keepdims=True)


@jax.jit
def kernel(x, w_gate, w_noise):
    n, d = x.shape
    _, k = w_gate.shape
    out_t = pl.pallas_call(
        _body,
        grid=(n // _BM,),
        in_specs=[
            pl.BlockSpec((_BM, d), lambda i: (i, 0)),
            pl.BlockSpec((d, k), lambda i: (0, 0)),
        ],
        out_specs=pl.BlockSpec((k, _BM), lambda i: (0, i)),
        out_shape=jax.ShapeDtypeStruct((k, n), jnp.float32),
        compiler_params=pltpu.CompilerParams(
            dimension_semantics=("arbitrary",),
        ),
    )(x, w_gate)
    return out_t.T
